# Initial kernel scaffold; baseline (speedup 1.0000x reference)
import jax
import jax.numpy as jnp
from jax.experimental import pallas as pl


def kernel(xs, h0, w_i2h, b_i2h, w_i2o, b_i2o):
    raise NotImplementedError("write your pallas kernel here")


# speedup vs baseline: 1.1500x; 1.1500x over previous
import functools

import jax
import jax.numpy as jnp
from jax.experimental import pallas as pl
from jax.experimental.pallas import tpu as pltpu

_LANE = 128
_SUB = 16


def _round_up(n, m):
    return ((n + m - 1) // m) * m


def _make_scan_body(Tc, T, I, H, BT):
    mask_tail = (T % Tc) != 0
    f32, bf16 = jnp.float32, jnp.bfloat16

    def body(x_ref, h0_ref, wxh_ref, whh_ref, bh_ref, hprev_ref, hfin_ref):
        tc = pl.program_id(1)

        @pl.when(tc == 0)
        def _():
            hfin_ref[...] = h0_ref[...]

        x = x_ref[...].reshape(Tc * BT, I)
        xc = (jnp.dot(x, wxh_ref[...], preferred_element_type=f32)
              + bh_ref[...]).reshape(Tc, BT, H)
        whh = whh_ref[...]
        t0 = tc * Tc

        h = hfin_ref[...]
        for t in range(Tc):
            hb = h.astype(bf16)
            hprev_ref[t] = hb
            h_new = xc[t] + jnp.dot(hb, whh, preferred_element_type=f32)
            if mask_tail:
                h_new = jnp.where(t0 + t < T, h_new, h)
            h = h_new
        hfin_ref[...] = h

    return body


def _make_out_body(Op):
    f32 = jnp.float32

    def body(x_ref, hp_ref, wxo_ref, who_ref, bo_ref, out_ref):
        logits = (jnp.dot(x_ref[...], wxo_ref[...], preferred_element_type=f32)
                  + jnp.dot(hp_ref[...], who_ref[...],
                            preferred_element_type=f32)
                  + bo_ref[...])
        m = jnp.max(logits, axis=1, keepdims=True)
        s = logits - m
        lse = jnp.log(jnp.sum(jnp.exp(s), axis=1, keepdims=True))
        out_ref[...] = s - lse

    return body


@functools.partial(jax.jit, static_argnames=())
def kernel(xs, h0, w_i2h, b_i2h, w_i2o, b_i2o):
    T, B, I = xs.shape
    H = w_i2h.shape[1]
    O = w_i2o.shape[1]
    f32, bf16 = jnp.float32, jnp.bfloat16

    Ip = _round_up(I, _LANE)
    Hp = _round_up(H, _LANE)
    Op = _round_up(O, _LANE)

    Bp = _round_up(B, _SUB)
    if Bp > _SUB:
        BT = min(64, _round_up(Bp // 2, _SUB))
        Bp = _round_up(Bp, BT)
    else:
        BT = Bp
    nbt = Bp // BT

    Tc = max(1, min(32, T))
    Tp = _round_up(T, Tc)
    ntc = Tp // Tc

    wxh = jnp.zeros((Ip, Hp), f32).at[:I, :H].set(w_i2h[:I].astype(f32)).astype(bf16)
    whh = jnp.zeros((Hp, Hp), f32).at[:H, :H].set(w_i2h[I:].astype(f32)).astype(bf16)
    wxo = jnp.zeros((Ip, Op), f32).at[:I, :O].set(w_i2o[:I].astype(f32)).astype(bf16)
    who = jnp.zeros((Hp, Op), f32).at[:H, :O].set(w_i2o[I:].astype(f32)).astype(bf16)
    bh = jnp.zeros((1, Hp), f32).at[:, :H].set(b_i2h.reshape(1, H).astype(f32))
    bo = jnp.zeros((1, Op), f32).at[:, :O].set(b_i2o.reshape(1, O).astype(f32))
    bo = bo.at[:, O:].set(-1e30)

    xs_p = jnp.zeros((Tp, Bp, Ip), bf16).at[:T, :B, :I].set(xs.astype(bf16))
    h0_p = jnp.zeros((Bp, Hp), f32).at[:B, :H].set(h0.astype(f32))

    resident = pl.Buffered(buffer_count=1)

    scan_cost = pl.CostEstimate(
        flops=2 * Tp * Bp * (Ip + Hp) * Hp,
        transcendentals=0,
        bytes_accessed=int(xs_p.size * 2 + Tp * Bp * Hp * 2
                           + (Ip + Hp) * Hp * 2 + Bp * Hp * 8))
    hprev, h_final = pl.pallas_call(
        _make_scan_body(Tc, T, Ip, Hp, BT),
        out_shape=(
            jax.ShapeDtypeStruct((Tp, Bp, Hp), bf16),
            jax.ShapeDtypeStruct((Bp, Hp), f32),
        ),
        grid=(nbt, ntc),
        in_specs=[
            pl.BlockSpec((Tc, BT, Ip), lambda bt, tc: (tc, bt, 0)),
            pl.BlockSpec((BT, Hp), lambda bt, tc: (bt, 0)),
            pl.BlockSpec((Ip, Hp), lambda bt, tc: (0, 0),
                         pipeline_mode=resident),
            pl.BlockSpec((Hp, Hp), lambda bt, tc: (0, 0),
                         pipeline_mode=resident),
            pl.BlockSpec((1, Hp), lambda bt, tc: (0, 0),
                         pipeline_mode=resident),
        ],
        out_specs=(
            pl.BlockSpec((Tc, BT, Hp), lambda bt, tc: (tc, bt, 0)),
            pl.BlockSpec((BT, Hp), lambda bt, tc: (bt, 0)),
        ),
        compiler_params=pltpu.CompilerParams(
            dimension_semantics=("parallel", "arbitrary"),
            vmem_limit_bytes=64 << 20),
        cost_estimate=scan_cost,
    )(xs_p, h0_p, wxh, whh, bh)

    R = Tp * Bp
    BR = min(2048, R)
    nr = R // BR
    xs2 = xs_p.reshape(R, Ip)
    hp2 = hprev.reshape(R, Hp)

    out_cost = pl.CostEstimate(
        flops=2 * R * (Ip + Hp) * Op,
        transcendentals=R * (Op + 1),
        bytes_accessed=int(R * (Ip + Hp) * 2 + R * Op * 4
                           + (Ip + Hp) * Op * 2))
    out = pl.pallas_call(
        _make_out_body(Op),
        out_shape=jax.ShapeDtypeStruct((R, Op), f32),
        grid=(nr,),
        in_specs=[
            pl.BlockSpec((BR, Ip), lambda r: (r, 0)),
            pl.BlockSpec((BR, Hp), lambda r: (r, 0)),
            pl.BlockSpec((Ip, Op), lambda r: (0, 0), pipeline_mode=resident),
            pl.BlockSpec((Hp, Op), lambda r: (0, 0), pipeline_mode=resident),
            pl.BlockSpec((1, Op), lambda r: (0, 0), pipeline_mode=resident),
        ],
        out_specs=pl.BlockSpec((BR, Op), lambda r: (r, 0)),
        compiler_params=pltpu.CompilerParams(
            dimension_semantics=("parallel",),
            vmem_limit_bytes=64 << 20),
        cost_estimate=out_cost,
    )(xs2, hp2, wxo, who, bo)

    out = out.reshape(Tp, Bp, Op)[:T, :B, :O]
    return out, h_final[:B, :H]


# speedup vs baseline: 1.3350x; 1.1609x over previous
import functools

import jax
import jax.numpy as jnp
from jax.experimental import pallas as pl
from jax.experimental.pallas import tpu as pltpu

_LANE = 128
_SUB = 8


def _round_up(n, m):
    return ((n + m - 1) // m) * m


def _make_body(Tc, T, I, H, O, BT):
    mask_tail = (T % Tc) != 0
    f32, bf16 = jnp.float32, jnp.bfloat16

    def body(x_ref, h0_ref, wih_ref, wio_ref, bh_ref, bo_ref,
             out_ref, hfin_ref, wih_s, wio_s, hp_s):
        tc = pl.program_id(1)

        @pl.when(tc == 0)
        def _():
            hfin_ref[...] = h0_ref[...]
            wih_s[...] = wih_ref[...].astype(bf16)
            wio_s[...] = wio_ref[...].astype(bf16)

        xb = x_ref[...].astype(bf16).reshape(Tc * BT, I)
        xc = (jnp.dot(xb, wih_s[:I], preferred_element_type=f32)
              + bh_ref[...]).reshape(Tc, BT, H)
        whh = wih_s[I:]
        t0 = tc * Tc

        h = hfin_ref[...]
        for t in range(Tc):
            hb = h.astype(bf16)
            hp_s[t] = hb
            h_new = xc[t] + jnp.dot(hb, whh, preferred_element_type=f32)
            if mask_tail:
                h_new = jnp.where(t0 + t < T, h_new, h)
            h = h_new
        hfin_ref[...] = h

        hp = hp_s[...].reshape(Tc * BT, H)
        logits = (jnp.dot(xb, wio_s[:I], preferred_element_type=f32)
                  + jnp.dot(hp, wio_s[I:], preferred_element_type=f32)
                  + bo_ref[...])
        m = jnp.max(logits, axis=1, keepdims=True)
        s = logits - m
        lse = jnp.log(jnp.sum(jnp.exp(s), axis=1, keepdims=True))
        out_ref[...] = (s - lse).reshape(Tc, BT, O)

    return body


@functools.partial(jax.jit, static_argnames=())
def kernel(xs, h0, w_i2h, b_i2h, w_i2o, b_i2o):
    T, B, I = xs.shape
    H = w_i2h.shape[1]
    O = w_i2o.shape[1]
    f32 = jnp.float32

    Ip, Hp, Op = (_round_up(d, _LANE) for d in (I, H, O))
    Bp = _round_up(B, _SUB)
    if Bp > 64:
        BT = 64
        Bp = _round_up(Bp, BT)
    else:
        BT = Bp
    nbt = Bp // BT
    Tc = max(1, min(32, T))
    Tp = _round_up(T, Tc)
    ntc = Tp // Tc

    pad = (Ip, Hp, Op, Bp, Tp) != (I, H, O, B, T)
    if pad:
        xs_in = jnp.zeros((Tp, Bp, Ip), f32).at[:T, :B, :I].set(xs.astype(f32))
        h0_in = jnp.zeros((Bp, Hp), f32).at[:B, :H].set(h0.astype(f32))
        wih = jnp.zeros((Ip + Hp, Hp), f32)
        wih = wih.at[:I, :H].set(w_i2h[:I].astype(f32))
        wih = wih.at[Ip:Ip + H, :H].set(w_i2h[I:].astype(f32))
        wio = jnp.zeros((Ip + Hp, Op), f32)
        wio = wio.at[:I, :O].set(w_i2o[:I].astype(f32))
        wio = wio.at[Ip:Ip + H, :O].set(w_i2o[I:].astype(f32))
        bh = jnp.zeros((1, Hp), f32).at[:, :H].set(b_i2h.reshape(1, H).astype(f32))
        bo = jnp.zeros((1, Op), f32).at[:, :O].set(b_i2o.reshape(1, O).astype(f32))
        bo = bo.at[:, O:].set(-1e30)
    else:
        xs_in, h0_in, wih, wio = xs, h0.astype(f32), w_i2h, w_i2o
        bh = b_i2h.reshape(1, H).astype(f32)
        bo = b_i2o.reshape(1, O).astype(f32)

    bf16 = jnp.bfloat16
    resident = pl.Buffered(buffer_count=1)
    cost = pl.CostEstimate(
        flops=2 * Tp * Bp * (Ip + Hp) * (Hp + Op),
        transcendentals=Tp * Bp * (Op + 1),
        bytes_accessed=int(Tp * Bp * Ip * 4 + Tp * Bp * Op * 4
                           + (Ip + Hp) * (Hp + Op) * 4 + Bp * Hp * 8))

    out, h_final = pl.pallas_call(
        _make_body(Tc, T, Ip, Hp, Op, BT),
        out_shape=(
            jax.ShapeDtypeStruct((Tp, Bp, Op), f32),
            jax.ShapeDtypeStruct((Bp, Hp), f32),
        ),
        grid=(nbt, ntc),
        in_specs=[
            pl.BlockSpec((Tc, BT, Ip), lambda bt, tc: (tc, bt, 0)),
            pl.BlockSpec((BT, Hp), lambda bt, tc: (bt, 0)),
            pl.BlockSpec((Ip + Hp, Hp), lambda bt, tc: (0, 0),
                         pipeline_mode=resident),
            pl.BlockSpec((Ip + Hp, Op), lambda bt, tc: (0, 0),
                         pipeline_mode=resident),
            pl.BlockSpec((1, Hp), lambda bt, tc: (0, 0),
                         pipeline_mode=resident),
            pl.BlockSpec((1, Op), lambda bt, tc: (0, 0),
                         pipeline_mode=resident),
        ],
        out_specs=(
            pl.BlockSpec((Tc, BT, Op), lambda bt, tc: (tc, bt, 0)),
            pl.BlockSpec((BT, Hp), lambda bt, tc: (bt, 0)),
        ),
        scratch_shapes=[
            pltpu.VMEM((Ip + Hp, Hp), bf16),
            pltpu.VMEM((Ip + Hp, Op), bf16),
            pltpu.VMEM((Tc, BT, Hp), bf16),
        ],
        compiler_params=pltpu.CompilerParams(
            dimension_semantics=("parallel", "arbitrary"),
            vmem_limit_bytes=100 << 20),
        cost_estimate=cost,
    )(xs_in, h0_in, wih, wio, bh, bo)

    if pad:
        return out[:T, :B, :O], h_final[:B, :H]
    return out, h_final
